# baseline (speedup 1.0000x reference)
import functools
import jax
import jax.numpy as jnp
from jax import lax
from jax.experimental import pallas as pl
from jax.experimental.pallas import tpu as pltpu
from jax.experimental.pallas import tpu_sc as plsc

N = 10000
E = 320000
D = 128
C = 128
NCHUNK = E // C
NW = 32
CHUNKS_PER_W = NCHUNK // NW
CHUNKS_REM = NCHUNK - CHUNKS_PER_W * NW
SLAB = 640
SLAB_TAIL = N - 15 * SLAB
N4 = 4 * N
N4P = 40960
DEG_PER_TILE = N4P // 16

_mesh = plsc.VectorSubcoreMesh(core_axis_name="c", subcore_axis_name="s")
_sc_params = pltpu.CompilerParams(needs_layout_passes=False)


KSUP = 5
NSUPTOT = NCHUNK // KSUP
SUP_PER_W = NSUPTOT // NW
SUP_REM = NSUPTOT - SUP_PER_W * NW


@functools.partial(
    pl.kernel,
    out_type=jax.ShapeDtypeStruct((2, N4P), jnp.float32),
    mesh=_mesh,
    scratch_types=[
        pltpu.VMEM_SHARED((N4P,), jnp.float32),
        pltpu.VMEM((KSUP, C), jnp.int32),
        pltpu.VMEM((C,), jnp.float32),
        pltpu.VMEM((DEG_PER_TILE,), jnp.float32),
        pltpu.SemaphoreType.DMA,
    ],
    compiler_params=_sc_params,
)
def _deg_kernel(gidx_hbm, degp_hbm, dacc, gi_v, ones_v, zbuf, sem):
    cid = lax.axis_index("c")
    sid = lax.axis_index("s")
    wid = sid * 2 + cid
    ones16 = jnp.ones((16,), jnp.float32)
    zeros16 = jnp.zeros((16,), jnp.float32)

    def fill(i, carry):
        zbuf[pl.ds(i * 16, 16)] = zeros16
        return carry

    lax.fori_loop(0, DEG_PER_TILE // 16, fill, 0)
    for j in range(C // 16):
        ones_v[pl.ds(j * 16, 16)] = ones16
    pltpu.sync_copy(zbuf, dacc.at[pl.ds(sid * DEG_PER_TILE, DEG_PER_TILE)])
    plsc.subcore_barrier()

    def super_chunk(sidx, carry):
        pltpu.sync_copy(gidx_hbm.at[sidx], gi_v)
        for k in range(KSUP):
            pltpu.async_copy(ones_v, dacc.at[gi_v.at[k]], sem, add=True)
        pltpu.make_async_copy(degp_hbm.at[0, pl.ds(0, KSUP * C)],
                              zbuf.at[pl.ds(0, KSUP * C)], sem).wait()
        return carry

    def sloop(s, carry):
        return super_chunk(wid * SUP_PER_W + s, carry)

    lax.fori_loop(0, SUP_PER_W, sloop, 0)

    @pl.when(wid < SUP_REM)
    def _():
        super_chunk(NW * SUP_PER_W + wid, 0)

    plsc.subcore_barrier()
    pltpu.sync_copy(dacc.at[pl.ds(sid * DEG_PER_TILE, DEG_PER_TILE)],
                    degp_hbm.at[cid, pl.ds(sid * DEG_PER_TILE, DEG_PER_TILE)])


def _k2_body(x_ref, w_ref, b_ref, root_ref, bond_ref, degp_ref,
             msg_ref, dis_ref, s_ref):
    x = x_ref[...]
    h = lax.dot_general(x, w_ref[...], (((1,), (1,)), ((), ())),
                        preferred_element_type=jnp.float32) + b_ref[...]
    deg = jnp.sum(degp_ref[...], axis=1) + 1.0
    dis = lax.rsqrt(deg)
    msg_ref[...] = (jnp.maximum(h[None, :, :] + bond_ref[...][:, None, :],
                                0.0) * dis[None, :, None])
    dis_ref[...] = dis[:, None]
    s_ref[...] = jnp.maximum(h + root_ref[...], 0.0) / deg[:, None]


_R2 = 2000

_k2 = pl.pallas_call(
    _k2_body,
    grid=(N // _R2,),
    in_specs=[
        pl.BlockSpec((_R2, D), lambda i: (i, 0)),
        pl.BlockSpec((D, D), lambda i: (0, 0)),
        pl.BlockSpec((1, D), lambda i: (0, 0)),
        pl.BlockSpec((1, D), lambda i: (0, 0)),
        pl.BlockSpec((4, D), lambda i: (0, 0)),
        pl.BlockSpec((_R2, 8), lambda i: (i, 0)),
    ],
    out_specs=[
        pl.BlockSpec((4, _R2, D), lambda i: (0, i, 0)),
        pl.BlockSpec((_R2, 1), lambda i: (i, 0)),
        pl.BlockSpec((_R2, D), lambda i: (i, 0)),
    ],
    out_shape=[
        jax.ShapeDtypeStruct((4, N, D), jnp.float32),
        jax.ShapeDtypeStruct((N, 1), jnp.float32),
        jax.ShapeDtypeStruct((N, D), jnp.float32),
    ],
)


@functools.partial(
    pl.kernel,
    out_type=jax.ShapeDtypeStruct((2, N, D), jnp.float32),
    mesh=_mesh,
    scratch_types=[
        pltpu.VMEM_SHARED((N, D), jnp.float32),
        pltpu.VMEM((2, C), jnp.int32),
        pltpu.VMEM((2, C), jnp.int32),
        pltpu.VMEM((2, C), jnp.int32),
        pltpu.VMEM((C, D), jnp.float32),
        pltpu.VMEM((C, D), jnp.float32),
        pltpu.VMEM((C, D), jnp.float32),
        pltpu.SemaphoreType.DMA,
        pltpu.SemaphoreType.DMA,
        pltpu.SemaphoreType.DMA,
        pltpu.SemaphoreType.DMA,
        pltpu.SemaphoreType.DMA,
        pltpu.SemaphoreType.DMA,
    ],
    compiler_params=_sc_params,
)
def _msg_kernel(edata_hbm, msg_hbm, out_hbm,
                acc, e0, e1, e2, r0, r1, r2, g0, g1, g2, s0, s1, s2):
    cid = lax.axis_index("c")
    sid = lax.axis_index("s")
    wid = sid * 2 + cid
    base_row = pl.multiple_of(sid * SLAB, 8)

    zeros16 = jnp.zeros((16,), jnp.float32)

    def zbody(i, carry):
        for j in range(D // 16):
            r0[i, pl.ds(j * 16, 16)] = zeros16
        return carry

    lax.fori_loop(0, C, zbody, 0)

    @pl.when(sid < 15)
    def _():
        for k in range(SLAB // C):
            pltpu.sync_copy(r0, acc.at[pl.ds(base_row + k * C, C)])

    @pl.when(sid == 15)
    def _():
        for k in range(SLAB_TAIL // C):
            pltpu.sync_copy(r0, acc.at[pl.ds(15 * SLAB + k * C, C)])
        rem = SLAB_TAIL - (SLAB_TAIL // C) * C
        if rem:
            pltpu.sync_copy(
                r0.at[pl.ds(0, rem)],
                acc.at[pl.ds(15 * SLAB + (SLAB_TAIL // C) * C, rem)])

    plsc.subcore_barrier()

    def fetch(t, e_b, r_b, gsem):
        g = wid + NW * t
        pltpu.sync_copy(edata_hbm.at[g], e_b)
        pltpu.async_copy(msg_hbm.at[e_b.at[0]], r_b, gsem)

    def wait_bytes(r_b, sem_b):
        pltpu.make_async_copy(msg_hbm.at[pl.ds(0, C)], r_b, sem_b).wait()

    NTRI = CHUNKS_PER_W // 3

    fetch(0, e0, r0, g0)
    fetch(1, e1, r1, g1)
    fetch(2, e2, r2, g2)

    def tri(q, carry):
        wait_bytes(r0, g0)
        pltpu.async_copy(r0, acc.at[e0.at[1]], s0, add=True)
        wait_bytes(r1, g1)
        pltpu.async_copy(r1, acc.at[e1.at[1]], s1, add=True)
        wait_bytes(r2, g2)
        pltpu.async_copy(r2, acc.at[e2.at[1]], s2, add=True)

        @pl.when(q < NTRI - 1)
        def _():
            wait_bytes(r0, s0)
            fetch(3 * q + 3, e0, r0, g0)
            wait_bytes(r1, s1)
            fetch(3 * q + 4, e1, r1, g1)
            wait_bytes(r2, s2)
            fetch(3 * q + 5, e2, r2, g2)

        return carry

    lax.fori_loop(0, NTRI, tri, 0)
    wait_bytes(r0, s0)
    wait_bytes(r1, s1)
    wait_bytes(r2, s2)

    @pl.when(wid < CHUNKS_REM)
    def _():
        fetch(CHUNKS_PER_W, e0, r0, g0)
        wait_bytes(r0, g0)
        pltpu.sync_copy(r0, acc.at[e0.at[1]], add=True)

    plsc.subcore_barrier()

    @pl.when(sid < 15)
    def _():
        pltpu.sync_copy(acc.at[pl.ds(base_row, SLAB)],
                        out_hbm.at[cid, pl.ds(base_row, SLAB)])

    @pl.when(sid == 15)
    def _():
        pltpu.sync_copy(acc.at[pl.ds(15 * SLAB, SLAB_TAIL)],
                        out_hbm.at[cid, pl.ds(15 * SLAB, SLAB_TAIL)])


def _k4_body(p_ref, dis_ref, s_ref, o_ref):
    o_ref[...] = (p_ref[0] + p_ref[1]) * dis_ref[...] + s_ref[...]


_k4 = pl.pallas_call(
    _k4_body,
    grid=(N // _R2,),
    in_specs=[
        pl.BlockSpec((2, _R2, D), lambda i: (0, i, 0)),
        pl.BlockSpec((_R2, 1), lambda i: (i, 0)),
        pl.BlockSpec((_R2, D), lambda i: (i, 0)),
    ],
    out_specs=pl.BlockSpec((_R2, D), lambda i: (i, 0)),
    out_shape=jax.ShapeDtypeStruct((N, D), jnp.float32),
)


def kernel(x, edge_index, edge_attr, W, b, root_emb, bond_table):
    row = edge_index[0]
    col = edge_index[1]
    gidx = edge_attr.astype(jnp.int32) * N + row
    edata = jnp.stack([gidx, col], axis=0).reshape(
        2, NCHUNK, C).transpose(1, 0, 2)

    degp = _deg_kernel(gidx.reshape(NSUPTOT, KSUP, C))
    degp8 = degp[:, :N4].reshape(8, N).T
    msgtab, dis2, selfterm = _k2(x, W, b.reshape(1, D), root_emb,
                                 bond_table, degp8)
    partials = _msg_kernel(edata, msgtab.reshape(N4, D))
    return _k4(partials, dis2, selfterm)


# speedup vs baseline: 1.0234x; 1.0234x over previous
import functools
import jax
import jax.numpy as jnp
from jax import lax
from jax.experimental import pallas as pl
from jax.experimental.pallas import tpu as pltpu
from jax.experimental.pallas import tpu_sc as plsc

N = 10000
E = 320000
D = 128
C = 128
NCHUNK = E // C
NW = 32
CHUNKS_PER_W = NCHUNK // NW
CHUNKS_REM = NCHUNK - CHUNKS_PER_W * NW
SLAB = 640
SLAB_TAIL = N - 15 * SLAB
N4 = 4 * N
N4P = 40960
DEG_PER_TILE = N4P // 16

_mesh = plsc.VectorSubcoreMesh(core_axis_name="c", subcore_axis_name="s")
_sc_params = pltpu.CompilerParams(needs_layout_passes=False)


KSUP = 10
NSUPTOT = NCHUNK // KSUP
SUP_PER_W = NSUPTOT // NW
SUP_REM = NSUPTOT - SUP_PER_W * NW


@functools.partial(
    pl.kernel,
    out_type=jax.ShapeDtypeStruct((2, N4P), jnp.float32),
    mesh=_mesh,
    scratch_types=[
        pltpu.VMEM_SHARED((N4P,), jnp.float32),
        pltpu.VMEM((KSUP, C), jnp.int32),
        pltpu.VMEM((C,), jnp.float32),
        pltpu.VMEM((DEG_PER_TILE,), jnp.float32),
        pltpu.SemaphoreType.DMA,
    ],
    compiler_params=_sc_params,
)
def _deg_kernel(gidx_hbm, degp_hbm, dacc, gi_v, ones_v, zbuf, sem):
    cid = lax.axis_index("c")
    sid = lax.axis_index("s")
    wid = sid * 2 + cid
    ones16 = jnp.ones((16,), jnp.float32)
    zeros16 = jnp.zeros((16,), jnp.float32)

    def fill(i, carry):
        zbuf[pl.ds(i * 16, 16)] = zeros16
        return carry

    lax.fori_loop(0, DEG_PER_TILE // 16, fill, 0)
    for j in range(C // 16):
        ones_v[pl.ds(j * 16, 16)] = ones16
    pltpu.sync_copy(zbuf, dacc.at[pl.ds(sid * DEG_PER_TILE, DEG_PER_TILE)])
    plsc.subcore_barrier()

    def super_chunk(sidx, carry):
        pltpu.sync_copy(gidx_hbm.at[sidx], gi_v)
        for k in range(KSUP):
            pltpu.async_copy(ones_v, dacc.at[gi_v.at[k]], sem, add=True)
        pltpu.make_async_copy(degp_hbm.at[0, pl.ds(0, KSUP * C)],
                              zbuf.at[pl.ds(0, KSUP * C)], sem).wait()
        return carry

    def sloop(s, carry):
        return super_chunk(wid * SUP_PER_W + s, carry)

    lax.fori_loop(0, SUP_PER_W, sloop, 0)

    @pl.when(wid < SUP_REM)
    def _():
        super_chunk(NW * SUP_PER_W + wid, 0)

    plsc.subcore_barrier()
    pltpu.sync_copy(dacc.at[pl.ds(sid * DEG_PER_TILE, DEG_PER_TILE)],
                    degp_hbm.at[cid, pl.ds(sid * DEG_PER_TILE, DEG_PER_TILE)])


def _k2_body(x_ref, w_ref, b_ref, root_ref, bond_ref, degp_ref,
             msg_ref, dis_ref, s_ref):
    x = x_ref[...]
    h = lax.dot_general(x, w_ref[...], (((1,), (1,)), ((), ())),
                        preferred_element_type=jnp.float32) + b_ref[...]
    deg = jnp.sum(degp_ref[...], axis=1) + 1.0
    dis = lax.rsqrt(deg)
    msg_ref[...] = (jnp.maximum(h[None, :, :] + bond_ref[...][:, None, :],
                                0.0) * dis[None, :, None])
    dis_ref[...] = dis[:, None]
    s_ref[...] = jnp.maximum(h + root_ref[...], 0.0) / deg[:, None]


_R2 = 2000

_k2 = pl.pallas_call(
    _k2_body,
    grid=(N // _R2,),
    in_specs=[
        pl.BlockSpec((_R2, D), lambda i: (i, 0)),
        pl.BlockSpec((D, D), lambda i: (0, 0)),
        pl.BlockSpec((1, D), lambda i: (0, 0)),
        pl.BlockSpec((1, D), lambda i: (0, 0)),
        pl.BlockSpec((4, D), lambda i: (0, 0)),
        pl.BlockSpec((_R2, 8), lambda i: (i, 0)),
    ],
    out_specs=[
        pl.BlockSpec((4, _R2, D), lambda i: (0, i, 0)),
        pl.BlockSpec((_R2, 1), lambda i: (i, 0)),
        pl.BlockSpec((_R2, D), lambda i: (i, 0)),
    ],
    out_shape=[
        jax.ShapeDtypeStruct((4, N, D), jnp.float32),
        jax.ShapeDtypeStruct((N, 1), jnp.float32),
        jax.ShapeDtypeStruct((N, D), jnp.float32),
    ],
)


@functools.partial(
    pl.kernel,
    out_type=jax.ShapeDtypeStruct((2, N, D), jnp.float32),
    mesh=_mesh,
    scratch_types=[
        pltpu.VMEM_SHARED((N, D), jnp.float32),
        pltpu.VMEM((2, C), jnp.int32),
        pltpu.VMEM((2, C), jnp.int32),
        pltpu.VMEM((2, C), jnp.int32),
        pltpu.VMEM((C, D), jnp.float32),
        pltpu.VMEM((C, D), jnp.float32),
        pltpu.VMEM((C, D), jnp.float32),
        pltpu.SemaphoreType.DMA,
        pltpu.SemaphoreType.DMA,
        pltpu.SemaphoreType.DMA,
        pltpu.SemaphoreType.DMA,
        pltpu.SemaphoreType.DMA,
        pltpu.SemaphoreType.DMA,
    ],
    compiler_params=_sc_params,
)
def _msg_kernel(edata_hbm, msg_hbm, out_hbm,
                acc, e0, e1, e2, r0, r1, r2, g0, g1, g2, s0, s1, s2):
    cid = lax.axis_index("c")
    sid = lax.axis_index("s")
    wid = sid * 2 + cid
    base_row = pl.multiple_of(sid * SLAB, 8)

    zeros16 = jnp.zeros((16,), jnp.float32)

    def zbody(i, carry):
        for j in range(D // 16):
            r0[i, pl.ds(j * 16, 16)] = zeros16
        return carry

    lax.fori_loop(0, C, zbody, 0)

    @pl.when(sid < 15)
    def _():
        for k in range(SLAB // C):
            pltpu.sync_copy(r0, acc.at[pl.ds(base_row + k * C, C)])

    @pl.when(sid == 15)
    def _():
        for k in range(SLAB_TAIL // C):
            pltpu.sync_copy(r0, acc.at[pl.ds(15 * SLAB + k * C, C)])
        rem = SLAB_TAIL - (SLAB_TAIL // C) * C
        if rem:
            pltpu.sync_copy(
                r0.at[pl.ds(0, rem)],
                acc.at[pl.ds(15 * SLAB + (SLAB_TAIL // C) * C, rem)])

    plsc.subcore_barrier()

    def fetch(t, e_b, r_b, gsem):
        g = wid + NW * t
        pltpu.sync_copy(edata_hbm.at[g], e_b)
        pltpu.async_copy(msg_hbm.at[e_b.at[0]], r_b, gsem)

    def wait_bytes(r_b, sem_b):
        pltpu.make_async_copy(msg_hbm.at[pl.ds(0, C)], r_b, sem_b).wait()

    NTRI = CHUNKS_PER_W // 3

    fetch(0, e0, r0, g0)
    fetch(1, e1, r1, g1)
    fetch(2, e2, r2, g2)

    def tri(q, carry):
        wait_bytes(r0, g0)
        pltpu.async_copy(r0, acc.at[e0.at[1]], s0, add=True)
        wait_bytes(r1, g1)
        pltpu.async_copy(r1, acc.at[e1.at[1]], s1, add=True)
        wait_bytes(r2, g2)
        pltpu.async_copy(r2, acc.at[e2.at[1]], s2, add=True)

        @pl.when(q < NTRI - 1)
        def _():
            wait_bytes(r0, s0)
            fetch(3 * q + 3, e0, r0, g0)
            wait_bytes(r1, s1)
            fetch(3 * q + 4, e1, r1, g1)
            wait_bytes(r2, s2)
            fetch(3 * q + 5, e2, r2, g2)

        return carry

    lax.fori_loop(0, NTRI, tri, 0)
    wait_bytes(r0, s0)
    wait_bytes(r1, s1)
    wait_bytes(r2, s2)

    @pl.when(wid < CHUNKS_REM)
    def _():
        fetch(CHUNKS_PER_W, e0, r0, g0)
        wait_bytes(r0, g0)
        pltpu.sync_copy(r0, acc.at[e0.at[1]], add=True)

    plsc.subcore_barrier()

    @pl.when(sid < 15)
    def _():
        pltpu.sync_copy(acc.at[pl.ds(base_row, SLAB)],
                        out_hbm.at[cid, pl.ds(base_row, SLAB)])

    @pl.when(sid == 15)
    def _():
        pltpu.sync_copy(acc.at[pl.ds(15 * SLAB, SLAB_TAIL)],
                        out_hbm.at[cid, pl.ds(15 * SLAB, SLAB_TAIL)])


def _k4_body(p_ref, dis_ref, s_ref, o_ref):
    o_ref[...] = (p_ref[0] + p_ref[1]) * dis_ref[...] + s_ref[...]


_k4 = pl.pallas_call(
    _k4_body,
    grid=(N // _R2,),
    in_specs=[
        pl.BlockSpec((2, _R2, D), lambda i: (0, i, 0)),
        pl.BlockSpec((_R2, 1), lambda i: (i, 0)),
        pl.BlockSpec((_R2, D), lambda i: (i, 0)),
    ],
    out_specs=pl.BlockSpec((_R2, D), lambda i: (i, 0)),
    out_shape=jax.ShapeDtypeStruct((N, D), jnp.float32),
)


def kernel(x, edge_index, edge_attr, W, b, root_emb, bond_table):
    row = edge_index[0]
    col = edge_index[1]
    gidx = edge_attr.astype(jnp.int32) * N + row
    edata = jnp.stack([gidx, col], axis=0).reshape(
        2, NCHUNK, C).transpose(1, 0, 2)

    degp = _deg_kernel(gidx.reshape(NSUPTOT, KSUP, C))
    degp8 = degp[:, :N4].reshape(8, N).T
    msgtab, dis2, selfterm = _k2(x, W, b.reshape(1, D), root_emb,
                                 bond_table, degp8)
    partials = _msg_kernel(edata, msgtab.reshape(N4, D))
    return _k4(partials, dis2, selfterm)


# speedup vs baseline: 1.0338x; 1.0102x over previous
import functools
import jax
import jax.numpy as jnp
from jax import lax
from jax.experimental import pallas as pl
from jax.experimental.pallas import tpu as pltpu
from jax.experimental.pallas import tpu_sc as plsc

N = 10000
E = 320000
D = 128
C = 128
NCHUNK = E // C
NW = 32
CHUNKS_PER_W = NCHUNK // NW
CHUNKS_REM = NCHUNK - CHUNKS_PER_W * NW
SLAB = 640
SLAB_TAIL = N - 15 * SLAB
N4 = 4 * N
N4P = 40960
DEG_PER_TILE = N4P // 16

_mesh = plsc.VectorSubcoreMesh(core_axis_name="c", subcore_axis_name="s")
_sc_params = pltpu.CompilerParams(needs_layout_passes=False)


KSUP = 20
NSUPTOT = NCHUNK // KSUP
SUP_PER_W = NSUPTOT // NW
SUP_REM = NSUPTOT - SUP_PER_W * NW


@functools.partial(
    pl.kernel,
    out_type=jax.ShapeDtypeStruct((2, N4P), jnp.float32),
    mesh=_mesh,
    scratch_types=[
        pltpu.VMEM_SHARED((N4P,), jnp.float32),
        pltpu.VMEM((KSUP, C), jnp.int32),
        pltpu.VMEM((C,), jnp.float32),
        pltpu.VMEM((DEG_PER_TILE,), jnp.float32),
        pltpu.SemaphoreType.DMA,
    ],
    compiler_params=_sc_params,
)
def _deg_kernel(gidx_hbm, degp_hbm, dacc, gi_v, ones_v, zbuf, sem):
    cid = lax.axis_index("c")
    sid = lax.axis_index("s")
    wid = sid * 2 + cid
    ones16 = jnp.ones((16,), jnp.float32)
    zeros16 = jnp.zeros((16,), jnp.float32)

    def fill(i, carry):
        zbuf[pl.ds(i * 16, 16)] = zeros16
        return carry

    lax.fori_loop(0, DEG_PER_TILE // 16, fill, 0)
    for j in range(C // 16):
        ones_v[pl.ds(j * 16, 16)] = ones16
    pltpu.sync_copy(zbuf, dacc.at[pl.ds(sid * DEG_PER_TILE, DEG_PER_TILE)])
    plsc.subcore_barrier()

    def super_chunk(sidx, carry):
        pltpu.sync_copy(gidx_hbm.at[sidx], gi_v)
        for k in range(KSUP):
            pltpu.async_copy(ones_v, dacc.at[gi_v.at[k]], sem, add=True)
        pltpu.make_async_copy(degp_hbm.at[0, pl.ds(0, KSUP * C)],
                              zbuf.at[pl.ds(0, KSUP * C)], sem).wait()
        return carry

    def sloop(s, carry):
        return super_chunk(wid * SUP_PER_W + s, carry)

    lax.fori_loop(0, SUP_PER_W, sloop, 0)

    @pl.when(wid < SUP_REM)
    def _():
        super_chunk(NW * SUP_PER_W + wid, 0)

    plsc.subcore_barrier()
    pltpu.sync_copy(dacc.at[pl.ds(sid * DEG_PER_TILE, DEG_PER_TILE)],
                    degp_hbm.at[cid, pl.ds(sid * DEG_PER_TILE, DEG_PER_TILE)])


def _k2_body(x_ref, w_ref, b_ref, root_ref, bond_ref, degp_ref,
             msg_ref, dis_ref, s_ref):
    x = x_ref[...]
    h = lax.dot_general(x, w_ref[...], (((1,), (1,)), ((), ())),
                        preferred_element_type=jnp.float32) + b_ref[...]
    deg = jnp.sum(degp_ref[...], axis=1) + 1.0
    dis = lax.rsqrt(deg)
    msg_ref[...] = (jnp.maximum(h[None, :, :] + bond_ref[...][:, None, :],
                                0.0) * dis[None, :, None])
    dis_ref[...] = dis[:, None]
    s_ref[...] = jnp.maximum(h + root_ref[...], 0.0) / deg[:, None]


_R2 = 2000

_k2 = pl.pallas_call(
    _k2_body,
    grid=(N // _R2,),
    in_specs=[
        pl.BlockSpec((_R2, D), lambda i: (i, 0)),
        pl.BlockSpec((D, D), lambda i: (0, 0)),
        pl.BlockSpec((1, D), lambda i: (0, 0)),
        pl.BlockSpec((1, D), lambda i: (0, 0)),
        pl.BlockSpec((4, D), lambda i: (0, 0)),
        pl.BlockSpec((_R2, 8), lambda i: (i, 0)),
    ],
    out_specs=[
        pl.BlockSpec((4, _R2, D), lambda i: (0, i, 0)),
        pl.BlockSpec((_R2, 1), lambda i: (i, 0)),
        pl.BlockSpec((_R2, D), lambda i: (i, 0)),
    ],
    out_shape=[
        jax.ShapeDtypeStruct((4, N, D), jnp.float32),
        jax.ShapeDtypeStruct((N, 1), jnp.float32),
        jax.ShapeDtypeStruct((N, D), jnp.float32),
    ],
)


@functools.partial(
    pl.kernel,
    out_type=jax.ShapeDtypeStruct((2, N, D), jnp.float32),
    mesh=_mesh,
    scratch_types=[
        pltpu.VMEM_SHARED((N, D), jnp.float32),
        pltpu.VMEM((2, C), jnp.int32),
        pltpu.VMEM((2, C), jnp.int32),
        pltpu.VMEM((2, C), jnp.int32),
        pltpu.VMEM((C, D), jnp.float32),
        pltpu.VMEM((C, D), jnp.float32),
        pltpu.VMEM((C, D), jnp.float32),
        pltpu.SemaphoreType.DMA,
        pltpu.SemaphoreType.DMA,
        pltpu.SemaphoreType.DMA,
        pltpu.SemaphoreType.DMA,
        pltpu.SemaphoreType.DMA,
        pltpu.SemaphoreType.DMA,
    ],
    compiler_params=_sc_params,
)
def _msg_kernel(edata_hbm, msg_hbm, out_hbm,
                acc, e0, e1, e2, r0, r1, r2, g0, g1, g2, s0, s1, s2):
    cid = lax.axis_index("c")
    sid = lax.axis_index("s")
    wid = sid * 2 + cid
    base_row = pl.multiple_of(sid * SLAB, 8)

    zeros16 = jnp.zeros((16,), jnp.float32)

    def zbody(i, carry):
        for j in range(D // 16):
            r0[i, pl.ds(j * 16, 16)] = zeros16
        return carry

    lax.fori_loop(0, C, zbody, 0)

    @pl.when(sid < 15)
    def _():
        for k in range(SLAB // C):
            pltpu.sync_copy(r0, acc.at[pl.ds(base_row + k * C, C)])

    @pl.when(sid == 15)
    def _():
        for k in range(SLAB_TAIL // C):
            pltpu.sync_copy(r0, acc.at[pl.ds(15 * SLAB + k * C, C)])
        rem = SLAB_TAIL - (SLAB_TAIL // C) * C
        if rem:
            pltpu.sync_copy(
                r0.at[pl.ds(0, rem)],
                acc.at[pl.ds(15 * SLAB + (SLAB_TAIL // C) * C, rem)])

    plsc.subcore_barrier()

    def fetch(t, e_b, r_b, gsem):
        g = wid + NW * t
        pltpu.sync_copy(edata_hbm.at[g], e_b)
        pltpu.async_copy(msg_hbm.at[e_b.at[0]], r_b, gsem)

    def wait_bytes(r_b, sem_b):
        pltpu.make_async_copy(msg_hbm.at[pl.ds(0, C)], r_b, sem_b).wait()

    NTRI = CHUNKS_PER_W // 3

    fetch(0, e0, r0, g0)
    fetch(1, e1, r1, g1)
    fetch(2, e2, r2, g2)

    def tri(q, carry):
        wait_bytes(r0, g0)
        pltpu.async_copy(r0, acc.at[e0.at[1]], s0, add=True)
        wait_bytes(r1, g1)
        pltpu.async_copy(r1, acc.at[e1.at[1]], s1, add=True)
        wait_bytes(r2, g2)
        pltpu.async_copy(r2, acc.at[e2.at[1]], s2, add=True)

        @pl.when(q < NTRI - 1)
        def _():
            wait_bytes(r0, s0)
            fetch(3 * q + 3, e0, r0, g0)
            wait_bytes(r1, s1)
            fetch(3 * q + 4, e1, r1, g1)
            wait_bytes(r2, s2)
            fetch(3 * q + 5, e2, r2, g2)

        return carry

    lax.fori_loop(0, NTRI, tri, 0)
    wait_bytes(r0, s0)
    wait_bytes(r1, s1)
    wait_bytes(r2, s2)

    @pl.when(wid < CHUNKS_REM)
    def _():
        fetch(CHUNKS_PER_W, e0, r0, g0)
        wait_bytes(r0, g0)
        pltpu.sync_copy(r0, acc.at[e0.at[1]], add=True)

    plsc.subcore_barrier()

    @pl.when(sid < 15)
    def _():
        pltpu.sync_copy(acc.at[pl.ds(base_row, SLAB)],
                        out_hbm.at[cid, pl.ds(base_row, SLAB)])

    @pl.when(sid == 15)
    def _():
        pltpu.sync_copy(acc.at[pl.ds(15 * SLAB, SLAB_TAIL)],
                        out_hbm.at[cid, pl.ds(15 * SLAB, SLAB_TAIL)])


def _k4_body(p_ref, dis_ref, s_ref, o_ref):
    o_ref[...] = (p_ref[0] + p_ref[1]) * dis_ref[...] + s_ref[...]


_k4 = pl.pallas_call(
    _k4_body,
    grid=(N // _R2,),
    in_specs=[
        pl.BlockSpec((2, _R2, D), lambda i: (0, i, 0)),
        pl.BlockSpec((_R2, 1), lambda i: (i, 0)),
        pl.BlockSpec((_R2, D), lambda i: (i, 0)),
    ],
    out_specs=pl.BlockSpec((_R2, D), lambda i: (i, 0)),
    out_shape=jax.ShapeDtypeStruct((N, D), jnp.float32),
)


def kernel(x, edge_index, edge_attr, W, b, root_emb, bond_table):
    row = edge_index[0]
    col = edge_index[1]
    gidx = edge_attr.astype(jnp.int32) * N + row
    edata = jnp.stack([gidx, col], axis=0).reshape(
        2, NCHUNK, C).transpose(1, 0, 2)

    degp = _deg_kernel(gidx.reshape(NSUPTOT, KSUP, C))
    degp8 = degp[:, :N4].reshape(8, N).T
    msgtab, dis2, selfterm = _k2(x, W, b.reshape(1, D), root_emb,
                                 bond_table, degp8)
    partials = _msg_kernel(edata, msgtab.reshape(N4, D))
    return _k4(partials, dis2, selfterm)
